# Initial kernel scaffold; baseline (speedup 1.0000x reference)
import jax
import jax.numpy as jnp
from jax.experimental import pallas as pl


def kernel(node_features, edge_index, weight, w_ih, w_hh, b_ih, b_hh, net_w1, net_b1, net_w2, net_b2, ro_w1, ro_b1, ro_w2, ro_b2):
    raise NotImplementedError("write your pallas kernel here")


# speedup vs baseline: 7.7443x; 7.7443x over previous
import functools

import jax
import jax.numpy as jnp
from jax.experimental import pallas as pl
from jax.experimental.pallas import tpu as pltpu

B, N, K = 128, 400, 3
H = 256
L = 2
NDIM = 2


def _elu(v):
    return jnp.where(v > 0, v, jnp.exp(jnp.minimum(v, 0.0)) - 1.0)


def _body(x_ref, e_ref, w_ref, wih_ref, whh_ref, bih_ref, bhh_ref,
          nw1_ref, nb1_ref, nw2_ref, nb2_ref, rw1_ref, rb1_ref, rw2_ref,
          rb2_ref, out_ref):
    x = x_ref[0]
    e = e_ref[0]

    iota = jax.lax.broadcasted_iota(jnp.int32, (N, N), 1)
    a = (e[:, 0:1] == iota).astype(jnp.float32)
    for k in range(1, K):
        a = a + (e[:, k:k + 1] == iota).astype(jnp.float32)

    bih = bih_ref[0]
    bhh = bhh_ref[0]
    for i in range(L):
        m = jnp.dot(x, w_ref[i], preferred_element_type=jnp.float32)
        s = jnp.dot(a, m, preferred_element_type=jnp.float32)
        gi = jnp.dot(s, wih_ref[...], preferred_element_type=jnp.float32) + bih
        gh = jnp.dot(x, whh_ref[...], preferred_element_type=jnp.float32) + bhh
        r = jax.nn.sigmoid(gi[:, :H] + gh[:, :H])
        z = jax.nn.sigmoid(gi[:, H:2 * H] + gh[:, H:2 * H])
        n = jnp.tanh(gi[:, 2 * H:] + r * gh[:, 2 * H:])
        x = (1.0 - z) * n + z * x

    h1 = _elu(jnp.dot(x, nw1_ref[...], preferred_element_type=jnp.float32)
              + nb1_ref[0])
    h2 = _elu(jnp.dot(h1, nw2_ref[...], preferred_element_type=jnp.float32)
              + nb2_ref[0])
    pooled = jnp.sum(h2, axis=0, keepdims=True)
    r1 = _elu(jnp.dot(pooled, rw1_ref[...], preferred_element_type=jnp.float32)
              + rb1_ref[0])
    out = jnp.dot(r1, rw2_ref[...], preferred_element_type=jnp.float32) \
        + rb2_ref[0]
    out_ref[0] = out


@jax.jit
def kernel(node_features, edge_index, weight, w_ih, w_hh, b_ih, b_hh,
           net_w1, net_b1, net_w2, net_b2, ro_w1, ro_b1, ro_w2, ro_b2):
    wih_t = w_ih.T
    whh_t = w_hh.T
    nw1_t = net_w1.T
    nw2_t = net_w2.T
    rw1_t = ro_w1.T
    rw2_t = ro_w2.T
    bih = b_ih.reshape(1, 3 * H)
    bhh = b_hh.reshape(1, 3 * H)
    nb1 = net_b1.reshape(1, H)
    nb2 = net_b2.reshape(1, H)
    rb1 = ro_b1.reshape(1, H)
    rb2 = ro_b2.reshape(1, NDIM)

    rep = lambda shape: pl.BlockSpec(shape, lambda b: (0,) * len(shape))
    grid_spec = pl.GridSpec(
        grid=(B,),
        in_specs=[
            pl.BlockSpec((1, N, H), lambda b: (b, 0, 0)),
            pl.BlockSpec((1, N, K), lambda b: (b, 0, 0)),
            rep((L, H, H)),
            rep((H, 3 * H)),
            rep((H, 3 * H)),
            rep((1, 3 * H)),
            rep((1, 3 * H)),
            rep((H, H)),
            rep((1, H)),
            rep((H, H)),
            rep((1, H)),
            rep((H, H)),
            rep((1, H)),
            rep((H, NDIM)),
            rep((1, NDIM)),
        ],
        out_specs=pl.BlockSpec((1, 1, NDIM), lambda b: (b, 0, 0)),
    )
    out = pl.pallas_call(
        _body,
        grid_spec=grid_spec,
        out_shape=jax.ShapeDtypeStruct((B, 1, NDIM), jnp.float32),
        compiler_params=pltpu.CompilerParams(
            dimension_semantics=("arbitrary",),
        ),
    )(node_features, edge_index, weight, wih_t, whh_t, bih, bhh,
      nw1_t, nb1, nw2_t, nb2, rw1_t, rb1, rw2_t, rb2)
    return out.reshape(B, NDIM)


# speedup vs baseline: 11.5006x; 1.4850x over previous
import functools

import jax
import jax.numpy as jnp
from jax.experimental import pallas as pl
from jax.experimental.pallas import tpu as pltpu

B, N, K = 128, 400, 3
H = 256
L = 2
NDIM = 2
BB = 4


def _elu(v):
    return jnp.where(v > 0, v, jnp.exp(jnp.minimum(v, 0.0)) - 1.0)


def _body(x_ref, e_ref, w_ref, wih_ref, whh_ref, bih_ref, bhh_ref,
          nw1_ref, nb1_ref, nw2_ref, nb2_ref, rw1_ref, rb1_ref, rw2_ref,
          rb2_ref, out_ref):
    x = x_ref[...].reshape(BB * N, H)

    iota = jax.lax.broadcasted_iota(jnp.int32, (N, N), 1)
    adj = []
    for g in range(BB):
        e = e_ref[g]
        a = (e[:, 0:1] == iota).astype(jnp.float32)
        for k in range(1, K):
            a = a + (e[:, k:k + 1] == iota).astype(jnp.float32)
        adj.append(a)

    bih = bih_ref[0]
    bhh = bhh_ref[0]
    for i in range(L):
        m = jnp.dot(x, w_ref[i], preferred_element_type=jnp.float32)
        s = jnp.concatenate(
            [jnp.dot(adj[g], m[g * N:(g + 1) * N],
                     preferred_element_type=jnp.float32)
             for g in range(BB)], axis=0)
        gi = jnp.dot(s, wih_ref[...], preferred_element_type=jnp.float32) + bih
        gh = jnp.dot(x, whh_ref[...], preferred_element_type=jnp.float32) + bhh
        r = jax.nn.sigmoid(gi[:, :H] + gh[:, :H])
        z = jax.nn.sigmoid(gi[:, H:2 * H] + gh[:, H:2 * H])
        n = jnp.tanh(gi[:, 2 * H:] + r * gh[:, 2 * H:])
        x = (1.0 - z) * n + z * x

    h1 = _elu(jnp.dot(x, nw1_ref[...], preferred_element_type=jnp.float32)
              + nb1_ref[0])
    h2 = _elu(jnp.dot(h1, nw2_ref[...], preferred_element_type=jnp.float32)
              + nb2_ref[0])
    pooled = jnp.sum(h2.reshape(BB, N, H), axis=1)
    r1 = _elu(jnp.dot(pooled, rw1_ref[...], preferred_element_type=jnp.float32)
              + rb1_ref[0])
    out = jnp.dot(r1, rw2_ref[...], preferred_element_type=jnp.float32) \
        + rb2_ref[0]
    out_ref[...] = out.reshape(BB, 1, NDIM)


@jax.jit
def kernel(node_features, edge_index, weight, w_ih, w_hh, b_ih, b_hh,
           net_w1, net_b1, net_w2, net_b2, ro_w1, ro_b1, ro_w2, ro_b2):
    wih_t = w_ih.T
    whh_t = w_hh.T
    nw1_t = net_w1.T
    nw2_t = net_w2.T
    rw1_t = ro_w1.T
    rw2_t = ro_w2.T
    bih = b_ih.reshape(1, 3 * H)
    bhh = b_hh.reshape(1, 3 * H)
    nb1 = net_b1.reshape(1, H)
    nb2 = net_b2.reshape(1, H)
    rb1 = ro_b1.reshape(1, H)
    rb2 = ro_b2.reshape(1, NDIM)

    rep = lambda shape: pl.BlockSpec(shape, lambda b: (0,) * len(shape))
    grid_spec = pl.GridSpec(
        grid=(B // BB,),
        in_specs=[
            pl.BlockSpec((BB, N, H), lambda b: (b, 0, 0)),
            pl.BlockSpec((BB, N, K), lambda b: (b, 0, 0)),
            rep((L, H, H)),
            rep((H, 3 * H)),
            rep((H, 3 * H)),
            rep((1, 3 * H)),
            rep((1, 3 * H)),
            rep((H, H)),
            rep((1, H)),
            rep((H, H)),
            rep((1, H)),
            rep((H, H)),
            rep((1, H)),
            rep((H, NDIM)),
            rep((1, NDIM)),
        ],
        out_specs=pl.BlockSpec((BB, 1, NDIM), lambda b: (b, 0, 0)),
    )
    out = pl.pallas_call(
        _body,
        grid_spec=grid_spec,
        out_shape=jax.ShapeDtypeStruct((B, 1, NDIM), jnp.float32),
        compiler_params=pltpu.CompilerParams(
            dimension_semantics=("arbitrary",),
        ),
    )(node_features, edge_index, weight, wih_t, whh_t, bih, bhh,
      nw1_t, nb1, nw2_t, nb2, rw1_t, rb1, rw2_t, rb2)
    return out.reshape(B, NDIM)


# speedup vs baseline: 11.7559x; 1.0222x over previous
import functools

import jax
import jax.numpy as jnp
from jax.experimental import pallas as pl
from jax.experimental.pallas import tpu as pltpu

B, N, K = 128, 400, 3
H = 256
L = 2
NDIM = 2
BB = 4


def _elu(v):
    return jnp.where(v > 0, v, jnp.exp(jnp.minimum(v, 0.0)) - 1.0)


def _sigmoid(v):
    return 0.5 + 0.5 * jnp.tanh(0.5 * v)


def _body(x_ref, e_ref, wcat_ref, wih_ref, bih_ref, bhh_ref,
          nw1_ref, nb1_ref, nw2_ref, nb2_ref, rw1_ref, rb1_ref, rw2_ref,
          rb2_ref, out_ref):
    x = x_ref[...].reshape(BB * N, H)

    iota = jax.lax.broadcasted_iota(jnp.int32, (N, N), 1)
    adj = []
    for g in range(BB):
        e = e_ref[g]
        a = (e[:, 0:1] == iota).astype(jnp.float32)
        for k in range(1, K):
            a = a + (e[:, k:k + 1] == iota).astype(jnp.float32)
        adj.append(a)

    bih = bih_ref[0]
    bhh = bhh_ref[0]
    for i in range(L):
        mg = jnp.dot(x, wcat_ref[i], preferred_element_type=jnp.float32)
        m = mg[:, :H]
        gh = mg[:, H:] + bhh
        s = jnp.concatenate(
            [jnp.dot(adj[g], m[g * N:(g + 1) * N],
                     preferred_element_type=jnp.float32)
             for g in range(BB)], axis=0)
        gi = jnp.dot(s, wih_ref[...], preferred_element_type=jnp.float32) + bih
        r = _sigmoid(gi[:, :H] + gh[:, :H])
        z = _sigmoid(gi[:, H:2 * H] + gh[:, H:2 * H])
        n = jnp.tanh(gi[:, 2 * H:] + r * gh[:, 2 * H:])
        x = (1.0 - z) * n + z * x

    h1 = _elu(jnp.dot(x, nw1_ref[...], preferred_element_type=jnp.float32)
              + nb1_ref[0])
    h2 = _elu(jnp.dot(h1, nw2_ref[...], preferred_element_type=jnp.float32)
              + nb2_ref[0])
    pooled = jnp.sum(h2.reshape(BB, N, H), axis=1)
    r1 = _elu(jnp.dot(pooled, rw1_ref[...], preferred_element_type=jnp.float32)
              + rb1_ref[0])
    out = jnp.dot(r1, rw2_ref[...], preferred_element_type=jnp.float32) \
        + rb2_ref[0]
    out_ref[...] = out.reshape(BB, 1, NDIM)


@jax.jit
def kernel(node_features, edge_index, weight, w_ih, w_hh, b_ih, b_hh,
           net_w1, net_b1, net_w2, net_b2, ro_w1, ro_b1, ro_w2, ro_b2):
    wih_t = w_ih.T
    whh_t = w_hh.T
    wcat = jnp.concatenate(
        [weight, jnp.broadcast_to(whh_t[None], (L, H, 3 * H))], axis=2)
    nw1_t = net_w1.T
    nw2_t = net_w2.T
    rw1_t = ro_w1.T
    rw2_t = ro_w2.T
    bih = b_ih.reshape(1, 3 * H)
    bhh = b_hh.reshape(1, 3 * H)
    nb1 = net_b1.reshape(1, H)
    nb2 = net_b2.reshape(1, H)
    rb1 = ro_b1.reshape(1, H)
    rb2 = ro_b2.reshape(1, NDIM)

    rep = lambda shape: pl.BlockSpec(shape, lambda b: (0,) * len(shape))
    grid_spec = pl.GridSpec(
        grid=(B // BB,),
        in_specs=[
            pl.BlockSpec((BB, N, H), lambda b: (b, 0, 0)),
            pl.BlockSpec((BB, N, K), lambda b: (b, 0, 0)),
            rep((L, H, 4 * H)),
            rep((H, 3 * H)),
            rep((1, 3 * H)),
            rep((1, 3 * H)),
            rep((H, H)),
            rep((1, H)),
            rep((H, H)),
            rep((1, H)),
            rep((H, H)),
            rep((1, H)),
            rep((H, NDIM)),
            rep((1, NDIM)),
        ],
        out_specs=pl.BlockSpec((BB, 1, NDIM), lambda b: (b, 0, 0)),
    )
    out = pl.pallas_call(
        _body,
        grid_spec=grid_spec,
        out_shape=jax.ShapeDtypeStruct((B, 1, NDIM), jnp.float32),
        compiler_params=pltpu.CompilerParams(
            dimension_semantics=("arbitrary",),
        ),
    )(node_features, edge_index, wcat, wih_t, bih, bhh,
      nw1_t, nb1, nw2_t, nb2, rw1_t, rb1, rw2_t, rb2)
    return out.reshape(B, NDIM)
